# baseline (speedup 1.0000x reference)
import jax
import jax.numpy as jnp
from jax import lax
from jax.experimental import pallas as pl
from jax.experimental.pallas import tpu as pltpu
from jax.experimental.pallas import tpu_sc as plsc

N = 320000
D = 128
G = 1024
NC = 2
NS = 16
NW = NC * NS
SC_N = 256000
TC_N = N - SC_N
ROWS_PER_W = SC_N // NW
SCHUNK = 80
GFACT = 4
GCHUNK = SCHUNK * GFACT
NGATHER = ROWS_PER_W // GCHUNK
NCHUNK = ROWS_PER_W // SCHUNK
NV = ROWS_PER_W // 16
TCBLK = 1000
NTCB = TC_N // TCBLK
WIN = 256
GPT = G // NS


def _sc_segment_sum(x_hbm, ids1_hbm, sums_hbm, cnts_hbm,
                    flat_v, rows_a, rows_b, red_a, red_b, idx16_a, idx16_b,
                    ext16_v, ext64_v, first_v, last_v, cnt_v,
                    sem_a, sem_b, sem_s0, sem_s1, acc_sh):
    cid = lax.axis_index("c")
    sid = lax.axis_index("s")
    wid = cid * NS + sid

    iota = lax.iota(jnp.int32, 16)
    z = jnp.zeros((16,), jnp.float32)
    zeros16 = iota * 0

    def zero_row(i, _):
        for c in range(D // 16):
            rows_a[i, pl.ds(c * 16, 16)] = z
        return 0

    lax.fori_loop(0, GPT, zero_row, 0)
    pltpu.sync_copy(rows_a.at[pl.ds(0, GPT)], acc_sh.at[pl.ds(sid * GPT, GPT)])

    pltpu.sync_copy(ids1_hbm.at[pl.ds(wid * ROWS_PER_W, ROWS_PER_W)],
                    flat_v.at[pl.ds(16, ROWS_PER_W)])
    m1 = jnp.full((16,), -1, jnp.int32)
    flat_v[pl.ds(0, 16)] = m1
    flat_v[pl.ds(16 + ROWS_PER_W, 16)] = m1

    def init_fl(i, _):
        first_v[pl.ds(i * 16, 16)] = zeros16 + 1
        last_v[pl.ds(i * 16, 16)] = zeros16
        return 0

    lax.fori_loop(0, G // 16, init_fl, 0)

    def count_body(t, _):
        base = 16 + t * 16
        cur = flat_v[pl.ds(base, 16)]
        prev = flat_v[pl.ds(base - 1, 16)]
        nxt = flat_v[pl.ds(base + 1, 16)]
        pos = t * 16 + iota
        plsc.store_scatter(first_v, [cur], pos, mask=cur != prev)
        plsc.store_scatter(last_v, [cur], pos, mask=cur != nxt)
        return 0

    lax.fori_loop(0, NV, count_body, 0)

    def emit_counts(i, _):
        c = last_v[pl.ds(i * 16, 16)] - first_v[pl.ds(i * 16, 16)] + 1
        cnt_v[pl.ds(i * 16, 16)] = c.astype(jnp.float32)
        return 0

    lax.fori_loop(0, G // 16, emit_counts, 0)
    pltpu.sync_copy(cnt_v, cnts_hbm.at[wid])

    plsc.subcore_barrier()

    base0 = wid * ROWS_PER_W
    gsplat = jnp.full((16,), G, jnp.int32)
    lane0 = iota == 0

    def gather(g, buf, sem):
        return pltpu.async_copy(x_hbm.at[pl.ds(base0 + g * GCHUNK, GCHUNK)],
                                buf, sem)

    def reduce_sub(sub_start, buf, buf_base, parity):
        red = (red_a, red_b)[parity]
        idx16 = (idx16_a, idx16_b)[parity]
        sem_s = (sem_s0, sem_s1)[parity]
        pltpu.make_async_copy(red.at[pl.ds(0, 16)],
                              acc_sh.at[idx16.at[0]], sem_s).wait()
        idx16[0, pl.ds(0, 16)] = gsplat
        ext16_v[0, pl.ds(0, 16)] = gsplat
        for m in range(4):
            ext64_v[0, pl.ds(m * 16, 16)] = gsplat
        sub_end = sub_start + SCHUNK

        def wcond(c):
            s, _ = c
            return s < sub_end

        def wbody(c):
            s, p = c
            grp = flat_v[pl.ds(16 + (s // 16) * 16, 16)]
            id_s = jnp.sum(jnp.where(iota == s % 16, grp, 0))
            lgrp = last_v[pl.ds((id_s // 16) * 16, 16)]
            lg = jnp.sum(jnp.where(iota == id_s % 16, lgrp, 0))
            e = jnp.minimum(lg + 1, sub_end)

            def addrow(r, acc):
                off = r - buf_base
                return tuple(acc[c] + buf[off, pl.ds(c * 16, 16)]
                             for c in range(D // 16))

            acc = lax.fori_loop(s, e, addrow, (z,) * (D // 16))
            for c in range(D // 16):
                red[p, pl.ds(c * 16, 16)] = acc[c]
            idsplat = zeros16 + id_s
            psplat = zeros16 + p
            pm = psplat - 16
            plsc.store_scatter(idx16, [zeros16, psplat], idsplat,
                               mask=lane0 & (psplat < 16))
            plsc.store_scatter(ext16_v, [zeros16, pm], idsplat,
                               mask=lane0 & (psplat >= 16) & (psplat < 32))
            plsc.store_scatter(ext64_v, [zeros16, pm], idsplat,
                               mask=lane0 & (psplat >= 16))
            return (e, p + 1)

        _, n = lax.while_loop(wcond, wbody, (sub_start, sub_start * 0))

        pltpu.async_copy(red.at[pl.ds(0, 16)],
                         acc_sh.at[idx16.at[0]], sem_s, add=True)

        @pl.when((n > 16) & (n <= 32))
        def _():
            pltpu.sync_copy(red.at[pl.ds(16, 16)],
                            acc_sh.at[ext16_v.at[0]], add=True)

        @pl.when(n > 32)
        def _():
            pltpu.sync_copy(red.at[pl.ds(16, 64)],
                            acc_sh.at[ext64_v.at[0]], add=True)

    def process(g, buf):
        gbase = g * GCHUNK
        for k in range(GFACT):
            reduce_sub(gbase + k * SCHUNK, buf, gbase, k % 2)

    idx16_a[0, pl.ds(0, 16)] = gsplat
    idx16_b[0, pl.ds(0, 16)] = gsplat
    pltpu.async_copy(red_a.at[pl.ds(0, 16)], acc_sh.at[idx16_a.at[0]],
                     sem_s0, add=True)
    pltpu.async_copy(red_b.at[pl.ds(0, 16)], acc_sh.at[idx16_b.at[0]],
                     sem_s1, add=True)

    gather(0, rows_a, sem_a)

    def main_body(j, _):
        ga = 2 * j
        gb = 2 * j + 1
        pltpu.make_async_copy(x_hbm.at[pl.ds(0, GCHUNK)], rows_a, sem_a).wait()
        gather(gb, rows_b, sem_b)
        process(ga, rows_a)
        pltpu.make_async_copy(x_hbm.at[pl.ds(0, GCHUNK)], rows_b, sem_b).wait()

        @pl.when(gb + 1 < NGATHER)
        def _():
            gather(gb + 1, rows_a, sem_a)

        process(gb, rows_b)
        return 0

    lax.fori_loop(0, NGATHER // 2, main_body, 0)

    pltpu.make_async_copy(x_hbm.at[pl.ds(0, GCHUNK)], rows_a, sem_a).wait()
    process(NGATHER - 1, rows_a)

    pltpu.make_async_copy(red_a.at[pl.ds(0, 16)],
                          acc_sh.at[idx16_a.at[0]], sem_s0).wait()
    pltpu.make_async_copy(red_b.at[pl.ds(0, 16)],
                          acc_sh.at[idx16_b.at[0]], sem_s1).wait()

    plsc.subcore_barrier()

    pltpu.sync_copy(acc_sh.at[pl.ds(sid * GPT, GPT)],
                    sums_hbm.at[cid, pl.ds(sid * GPT, GPT)])


def _segment_sums(x, ids1d):
    mesh = plsc.VectorSubcoreMesh(core_axis_name="c", subcore_axis_name="s")
    return pl.kernel(
        _sc_segment_sum,
        out_type=(
            jax.ShapeDtypeStruct((NC, G, D), jnp.float32),
            jax.ShapeDtypeStruct((NW, G), jnp.float32),
        ),
        mesh=mesh,
        compiler_params=pltpu.CompilerParams(needs_layout_passes=False),
        scratch_types=[
            pltpu.VMEM((ROWS_PER_W + 32,), jnp.int32),
            pltpu.VMEM((GCHUNK, D), jnp.float32),
            pltpu.VMEM((GCHUNK, D), jnp.float32),
            pltpu.VMEM((SCHUNK, D), jnp.float32),
            pltpu.VMEM((SCHUNK, D), jnp.float32),
            pltpu.VMEM((1, 16), jnp.int32),
            pltpu.VMEM((1, 16), jnp.int32),
            pltpu.VMEM((1, 16), jnp.int32),
            pltpu.VMEM((1, 64), jnp.int32),
            pltpu.VMEM((G,), jnp.int32),
            pltpu.VMEM((G,), jnp.int32),
            pltpu.VMEM((G,), jnp.float32),
            pltpu.SemaphoreType.DMA,
            pltpu.SemaphoreType.DMA,
            pltpu.SemaphoreType.DMA,
            pltpu.SemaphoreType.DMA,
            pltpu.VMEM_SHARED((G + 8, D), jnp.float32),
        ],
    )(x, ids1d)


def _tc_sum_body(sb_ref, sp_ref, ids_ref, x_ref, os_ref, oc_ref,
                 accs_ref, accc_ref):
    b = pl.program_id(0)

    @pl.when(b == 0)
    def _():
        accs_ref[...] = jnp.zeros_like(accs_ref)
        accc_ref[...] = jnp.zeros_like(accc_ref)

    base = sb_ref[b]
    span = sp_ref[b]
    basea = (base // 8) * 8
    xf = x_ref[...]
    xh = xf.astype(jnp.bfloat16)
    xl = (xf - xh.astype(jnp.float32)).astype(jnp.bfloat16)
    ids = ids_ref[...]
    jrow = lax.broadcasted_iota(jnp.int32, (1, WIN), 1)
    ones8 = jnp.ones((TCBLK, 8), jnp.bfloat16)
    dn = (((0,), (0,)), ((), ()))
    for w in range(5):
        @pl.when((WIN * w) <= (span + 7))
        def _():
            rel = ids - basea - WIN * w
            oh = (rel == jrow).astype(jnp.bfloat16)
            sw = (lax.dot_general(oh, xh, dn, preferred_element_type=jnp.float32)
                  + lax.dot_general(oh, xl, dn, preferred_element_type=jnp.float32))
            cw = lax.dot_general(oh, ones8, dn, preferred_element_type=jnp.float32)
            st = basea + WIN * w
            accs_ref[pl.ds(st, WIN), :] += sw
            accc_ref[pl.ds(st, WIN), :] += cw

    @pl.when(b == NTCB - 1)
    def _():
        os_ref[...] = accs_ref[pl.ds(0, G), :]
        oc_ref[...] = accc_ref[pl.ds(0, G), :]


def _tc_partial(x2d, ids2d):
    grid_spec = pltpu.PrefetchScalarGridSpec(
        num_scalar_prefetch=2,
        grid=(NTCB,),
        in_specs=[
            pl.BlockSpec((TCBLK, 1), lambda b, sb, sp: (SC_N // TCBLK + b, 0)),
            pl.BlockSpec((TCBLK, D), lambda b, sb, sp: (SC_N // TCBLK + b, 0)),
        ],
        out_specs=[
            pl.BlockSpec((G, D), lambda b, sb, sp: (0, 0)),
            pl.BlockSpec((G, 8), lambda b, sb, sp: (0, 0)),
        ],
        scratch_shapes=[
            pltpu.VMEM((2048, D), jnp.float32),
            pltpu.VMEM((2048, 8), jnp.float32),
        ],
    )
    sbase = ids2d[SC_N // TCBLK:, 0]
    sspan = ids2d[SC_N // TCBLK:, TCBLK - 1] - sbase
    return pl.pallas_call(
        _tc_sum_body,
        grid_spec=grid_spec,
        out_shape=(
            jax.ShapeDtypeStruct((G, D), jnp.float32),
            jax.ShapeDtypeStruct((G, 8), jnp.float32),
        ),
    )(sbase, sspan, ids2d.reshape(N, 1), x2d)


def _mlp_body(ps_ref, pc_ref, ts_ref, tc_ref, w1_ref, b1_ref, w2_ref, b2_ref,
              o_ref):
    s = ps_ref[0] + ps_ref[1] + ts_ref[...]
    ones = jnp.ones((NW, 1), jnp.float32)
    c = lax.dot_general(pc_ref[...], ones, (((0,), (0,)), ((), ())),
                        precision=lax.Precision.HIGHEST,
                        preferred_element_type=jnp.float32) + tc_ref[:, 0:1]
    pooled = s / jnp.maximum(c, 1.0)
    h = jnp.maximum(
        jnp.dot(pooled, w1_ref[...], preferred_element_type=jnp.float32)
        + b1_ref[...], 0.0)
    o_ref[...] = (jnp.dot(h, w2_ref[...], preferred_element_type=jnp.float32)
                  + b2_ref[...])


def _mlp(psums, pcnts, tcs, tcc, W1, b1, W2, b2):
    return pl.pallas_call(
        _mlp_body,
        out_shape=jax.ShapeDtypeStruct((G, 1), jnp.float32),
    )(psums, pcnts, tcs, tcc, W1, b1.reshape(1, D), W2, b2.reshape(1, 1))


def kernel(x, segment_ids, y, W1, b1, W2, b2):
    ids1d = segment_ids.astype(jnp.int32)
    tcs, tcc = _tc_partial(x, ids1d.reshape(N // TCBLK, TCBLK))
    psums, pcnts = _segment_sums(x, ids1d)
    pred = _mlp(psums[:, :G], pcnts, tcs, tcc, W1, b1, W2, b2)
    return (pred, y)


# speedup vs baseline: 2.2845x; 2.2845x over previous
import jax
import jax.numpy as jnp
from jax import lax
from jax.experimental import pallas as pl
from jax.experimental.pallas import tpu as pltpu
from jax.experimental.pallas import tpu_sc as plsc

N = 320000
D = 128
G = 1024
NC = 2
NS = 16
NW = NC * NS
SC_N = 256000
TC_N = N - SC_N
ROWS_PER_W = SC_N // NW
SCHUNK = 80
GFACT = 4
GCHUNK = SCHUNK * GFACT
NGATHER = ROWS_PER_W // GCHUNK
NCHUNK = ROWS_PER_W // SCHUNK
NV = ROWS_PER_W // 16
TCBLK = 1000
NTCB = TC_N // TCBLK
WIN = 256
GPT = G // NS


def _sc_segment_sum(x_hbm, ids1_hbm, sums_hbm, cnts_hbm,
                    flat_v, rows_a, rows_b, red_a, red_b, idx16_a, idx16_b,
                    ext16_v, ext64_v, first_v, last_v, cnt_v,
                    sem_a, sem_b, sem_s0, sem_s1, acc_sh):
    cid = lax.axis_index("c")
    sid = lax.axis_index("s")
    wid = cid * NS + sid

    iota = lax.iota(jnp.int32, 16)
    z = jnp.zeros((16,), jnp.float32)
    zeros16 = iota * 0

    def zero_row(i, _):
        for c in range(D // 16):
            rows_a[i, pl.ds(c * 16, 16)] = z
        return 0

    lax.fori_loop(0, GPT, zero_row, 0)
    pltpu.sync_copy(rows_a.at[pl.ds(0, GPT)], acc_sh.at[pl.ds(sid * GPT, GPT)])

    pltpu.sync_copy(ids1_hbm.at[pl.ds(wid * ROWS_PER_W, ROWS_PER_W)],
                    flat_v.at[pl.ds(16, ROWS_PER_W)])
    m1 = jnp.full((16,), -1, jnp.int32)
    flat_v[pl.ds(0, 16)] = m1
    flat_v[pl.ds(16 + ROWS_PER_W, 16)] = m1

    def init_fl(i, _):
        first_v[pl.ds(i * 16, 16)] = zeros16 + 1
        last_v[pl.ds(i * 16, 16)] = zeros16
        return 0

    lax.fori_loop(0, G // 16, init_fl, 0)

    def count_body(t, _):
        base = 16 + t * 16
        cur = flat_v[pl.ds(base, 16)]
        prev = flat_v[pl.ds(base - 1, 16)]
        nxt = flat_v[pl.ds(base + 1, 16)]
        pos = t * 16 + iota
        plsc.store_scatter(first_v, [cur], pos, mask=cur != prev)
        plsc.store_scatter(last_v, [cur], pos, mask=cur != nxt)
        return 0

    lax.fori_loop(0, NV, count_body, 0)

    def emit_counts(i, _):
        c = last_v[pl.ds(i * 16, 16)] - first_v[pl.ds(i * 16, 16)] + 1
        cnt_v[pl.ds(i * 16, 16)] = c.astype(jnp.float32)
        return 0

    lax.fori_loop(0, G // 16, emit_counts, 0)
    pltpu.sync_copy(cnt_v, cnts_hbm.at[wid])

    plsc.subcore_barrier()

    base0 = wid * ROWS_PER_W
    gsplat = jnp.full((16,), G, jnp.int32)
    lane0 = iota == 0

    def gather(g, buf, sem):
        return pltpu.async_copy(x_hbm.at[pl.ds(base0 + g * GCHUNK, GCHUNK)],
                                buf, sem)

    def reduce_sub(sub_start, buf, buf_base, parity):
        red = (red_a, red_b)[parity]
        idx16 = (idx16_a, idx16_b)[parity]
        sem_s = (sem_s0, sem_s1)[parity]
        pltpu.make_async_copy(red.at[pl.ds(0, 16)],
                              acc_sh.at[idx16.at[0]], sem_s).wait()
        idx16[0, pl.ds(0, 16)] = gsplat
        ext16_v[0, pl.ds(0, 16)] = gsplat
        for m in range(4):
            ext64_v[0, pl.ds(m * 16, 16)] = gsplat
        sub_end = sub_start + SCHUNK

        def wcond(c):
            s, _ = c
            return s < sub_end

        def wbody(c):
            s, p = c
            grp = flat_v[pl.ds(16 + (s // 16) * 16, 16)]
            id_s = jnp.sum(jnp.where(iota == s % 16, grp, 0))
            lgrp = last_v[pl.ds((id_s // 16) * 16, 16)]
            lg = jnp.sum(jnp.where(iota == id_s % 16, lgrp, 0))
            e = jnp.minimum(lg + 1, sub_end)

            def addrow(r, acc):
                off = r - buf_base
                return tuple(acc[c] + buf[off, pl.ds(c * 16, 16)]
                             for c in range(D // 16))

            acc = lax.fori_loop(s, e, addrow, (z,) * (D // 16))
            for c in range(D // 16):
                red[p, pl.ds(c * 16, 16)] = acc[c]
            idsplat = zeros16 + id_s
            psplat = zeros16 + p
            pm = psplat - 16
            plsc.store_scatter(idx16, [zeros16, psplat], idsplat,
                               mask=lane0 & (psplat < 16))
            plsc.store_scatter(ext16_v, [zeros16, pm], idsplat,
                               mask=lane0 & (psplat >= 16) & (psplat < 32))
            plsc.store_scatter(ext64_v, [zeros16, pm], idsplat,
                               mask=lane0 & (psplat >= 16))
            return (e, p + 1)

        _, n = lax.while_loop(wcond, wbody, (sub_start, sub_start * 0))

        pltpu.async_copy(red.at[pl.ds(0, 16)],
                         acc_sh.at[idx16.at[0]], sem_s, add=True)

        @pl.when((n > 16) & (n <= 32))
        def _():
            pltpu.sync_copy(red.at[pl.ds(16, 16)],
                            acc_sh.at[ext16_v.at[0]], add=True)

        @pl.when(n > 32)
        def _():
            pltpu.sync_copy(red.at[pl.ds(16, 64)],
                            acc_sh.at[ext64_v.at[0]], add=True)

    def process(g, buf):
        gbase = g * GCHUNK
        for k in range(GFACT):
            reduce_sub(gbase + k * SCHUNK, buf, gbase, k % 2)

    idx16_a[0, pl.ds(0, 16)] = gsplat
    idx16_b[0, pl.ds(0, 16)] = gsplat
    pltpu.async_copy(red_a.at[pl.ds(0, 16)], acc_sh.at[idx16_a.at[0]],
                     sem_s0, add=True)
    pltpu.async_copy(red_b.at[pl.ds(0, 16)], acc_sh.at[idx16_b.at[0]],
                     sem_s1, add=True)

    gather(0, rows_a, sem_a)

    def main_body(j, _):
        ga = 2 * j
        gb = 2 * j + 1
        pltpu.make_async_copy(x_hbm.at[pl.ds(0, GCHUNK)], rows_a, sem_a).wait()
        gather(gb, rows_b, sem_b)
        process(ga, rows_a)
        pltpu.make_async_copy(x_hbm.at[pl.ds(0, GCHUNK)], rows_b, sem_b).wait()

        @pl.when(gb + 1 < NGATHER)
        def _():
            gather(gb + 1, rows_a, sem_a)

        process(gb, rows_b)
        return 0

    lax.fori_loop(0, NGATHER // 2, main_body, 0)

    pltpu.make_async_copy(x_hbm.at[pl.ds(0, GCHUNK)], rows_a, sem_a).wait()
    process(NGATHER - 1, rows_a)

    pltpu.make_async_copy(red_a.at[pl.ds(0, 16)],
                          acc_sh.at[idx16_a.at[0]], sem_s0).wait()
    pltpu.make_async_copy(red_b.at[pl.ds(0, 16)],
                          acc_sh.at[idx16_b.at[0]], sem_s1).wait()

    plsc.subcore_barrier()

    pltpu.sync_copy(acc_sh.at[pl.ds(sid * GPT, GPT)],
                    sums_hbm.at[cid, pl.ds(sid * GPT, GPT)])


def _segment_sums(x, ids1d):
    mesh = plsc.VectorSubcoreMesh(core_axis_name="c", subcore_axis_name="s")
    return pl.kernel(
        _sc_segment_sum,
        out_type=(
            jax.ShapeDtypeStruct((NC, G, D), jnp.float32),
            jax.ShapeDtypeStruct((NW, G), jnp.float32),
        ),
        mesh=mesh,
        compiler_params=pltpu.CompilerParams(needs_layout_passes=False),
        scratch_types=[
            pltpu.VMEM((ROWS_PER_W + 32,), jnp.int32),
            pltpu.VMEM((GCHUNK, D), jnp.float32),
            pltpu.VMEM((GCHUNK, D), jnp.float32),
            pltpu.VMEM((SCHUNK, D), jnp.float32),
            pltpu.VMEM((SCHUNK, D), jnp.float32),
            pltpu.VMEM((1, 16), jnp.int32),
            pltpu.VMEM((1, 16), jnp.int32),
            pltpu.VMEM((1, 16), jnp.int32),
            pltpu.VMEM((1, 64), jnp.int32),
            pltpu.VMEM((G,), jnp.int32),
            pltpu.VMEM((G,), jnp.int32),
            pltpu.VMEM((G,), jnp.float32),
            pltpu.SemaphoreType.DMA,
            pltpu.SemaphoreType.DMA,
            pltpu.SemaphoreType.DMA,
            pltpu.SemaphoreType.DMA,
            pltpu.VMEM_SHARED((G + 8, D), jnp.float32),
        ],
    )(x, ids1d)


def _tc_sum_body(sb_ref, sp_ref, ids_ref, x_ref, os_ref, oc_ref,
                 accs_ref, accc_ref):
    b = pl.program_id(0)

    @pl.when(b == 0)
    def _():
        accs_ref[...] = jnp.zeros_like(accs_ref)
        accc_ref[...] = jnp.zeros_like(accc_ref)

    base = sb_ref[b]
    span = sp_ref[b]
    basea = (base // 8) * 8
    xf = x_ref[...]
    xh = xf.astype(jnp.bfloat16)
    xl = (xf - xh.astype(jnp.float32)).astype(jnp.bfloat16)
    ids = ids_ref[0]
    jcol = lax.broadcasted_iota(jnp.int32, (WIN, 1), 0)
    ones8 = jnp.ones((TCBLK, 8), jnp.bfloat16)
    dn = (((1,), (0,)), ((), ()))
    for w in range(5):
        @pl.when((WIN * w) <= (span + 7))
        def _():
            rel = ids - basea - WIN * w
            oh = (rel == jcol).astype(jnp.bfloat16)
            sw = (lax.dot_general(oh, xh, dn, preferred_element_type=jnp.float32)
                  + lax.dot_general(oh, xl, dn, preferred_element_type=jnp.float32))
            cw = lax.dot_general(oh, ones8, dn, preferred_element_type=jnp.float32)
            st = basea + WIN * w
            accs_ref[pl.ds(st, WIN), :] += sw
            accc_ref[pl.ds(st, WIN), :] += cw

    @pl.when(b == NTCB - 1)
    def _():
        os_ref[...] = accs_ref[pl.ds(0, G), :]
        oc_ref[...] = accc_ref[pl.ds(0, G), :]


def _tc_partial(x2d, ids2d):
    grid_spec = pltpu.PrefetchScalarGridSpec(
        num_scalar_prefetch=2,
        grid=(NTCB,),
        in_specs=[
            pl.BlockSpec((1, 1, TCBLK), lambda b, sb, sp: (SC_N // TCBLK + b, 0, 0)),
            pl.BlockSpec((TCBLK, D), lambda b, sb, sp: (SC_N // TCBLK + b, 0)),
        ],
        out_specs=[
            pl.BlockSpec((G, D), lambda b, sb, sp: (0, 0)),
            pl.BlockSpec((G, 8), lambda b, sb, sp: (0, 0)),
        ],
        scratch_shapes=[
            pltpu.VMEM((2048, D), jnp.float32),
            pltpu.VMEM((2048, 8), jnp.float32),
        ],
    )
    sbase = ids2d[SC_N // TCBLK:, 0]
    sspan = ids2d[SC_N // TCBLK:, TCBLK - 1] - sbase
    return pl.pallas_call(
        _tc_sum_body,
        grid_spec=grid_spec,
        out_shape=(
            jax.ShapeDtypeStruct((G, D), jnp.float32),
            jax.ShapeDtypeStruct((G, 8), jnp.float32),
        ),
    )(sbase, sspan, ids2d.reshape(N // TCBLK, 1, TCBLK), x2d)


def _mlp_body(ps_ref, pc_ref, ts_ref, tc_ref, w1_ref, b1_ref, w2_ref, b2_ref,
              o_ref):
    s = ps_ref[0] + ps_ref[1] + ts_ref[...]
    ones = jnp.ones((NW, 1), jnp.float32)
    c = lax.dot_general(pc_ref[...], ones, (((0,), (0,)), ((), ())),
                        precision=lax.Precision.HIGHEST,
                        preferred_element_type=jnp.float32) + tc_ref[:, 0:1]
    pooled = s / jnp.maximum(c, 1.0)
    h = jnp.maximum(
        jnp.dot(pooled, w1_ref[...], preferred_element_type=jnp.float32)
        + b1_ref[...], 0.0)
    o_ref[...] = (jnp.dot(h, w2_ref[...], preferred_element_type=jnp.float32)
                  + b2_ref[...])


def _mlp(psums, pcnts, tcs, tcc, W1, b1, W2, b2):
    return pl.pallas_call(
        _mlp_body,
        out_shape=jax.ShapeDtypeStruct((G, 1), jnp.float32),
    )(psums, pcnts, tcs, tcc, W1, b1.reshape(1, D), W2, b2.reshape(1, 1))


def kernel(x, segment_ids, y, W1, b1, W2, b2):
    ids1d = segment_ids.astype(jnp.int32)
    tcs, tcc = _tc_partial(x, ids1d.reshape(N // TCBLK, TCBLK))
    psums, pcnts = _segment_sums(x, ids1d)
    pred = _mlp(psums[:, :G], pcnts, tcs, tcc, W1, b1, W2, b2)
    return (pred, y)
